# Initial kernel scaffold; baseline (speedup 1.0000x reference)
import jax
import jax.numpy as jnp
from jax.experimental import pallas as pl


def kernel(x, W1, b1, W2, b2, W3, b3, log_temp):
    raise NotImplementedError("write your pallas kernel here")


# speedup vs baseline: 22.8189x; 22.8189x over previous
import functools
import math

import numpy as np
import jax
import jax.numpy as jnp
from jax import lax
from jax.experimental import pallas as pl
from jax.experimental.pallas import tpu as pltpu
from jax.experimental.pallas import tpu_sc as plsc

_P = 10
_S = 5
_K = 7
_N1 = 15
_N2 = 15
_M = 225
_MP = 240
_C = 8
_H = 80
_W = 80
_HP = 82
_FLAT = _HP * _HP
_FLATP = 6728
_F = 800
_NEG = -1e30


def _build_patch_table():
    q = np.arange(100)
    ph, pw = q // 10, q % 10
    mcol = np.arange(_MP)
    m = np.where(mcol < _M, mcol, 0)
    i, j = m // _N2, m % _N2
    r = 5 * i[None, :] + ph[:, None]
    cc = 5 * j[None, :] + pw[:, None]
    return ((r + 1) * _HP + (cc + 1)).astype(np.int32)


def _build_fold_tables():
    p = np.arange(_H * _W)
    r, cc = p // _W, p % _W
    fidx = np.zeros((4, _H * _W), np.int32)
    wt = np.zeros((4, _H * _W), np.float32)
    vR = [(r // 5 - di >= 0) & (r // 5 - di <= 14) for di in (0, 1)]
    vC = [(cc // 5 - dj >= 0) & (cc // 5 - dj <= 14) for dj in (0, 1)]
    cntR = vR[0].astype(np.int32) + vR[1].astype(np.int32)
    cntC = vC[0].astype(np.int32) + vC[1].astype(np.int32)
    for t, (di, dj) in enumerate(((0, 0), (0, 1), (1, 0), (1, 1))):
        i = np.clip(r // 5 - di, 0, 14)
        j = np.clip(cc // 5 - dj, 0, 14)
        ph = r % 5 + 5 * di
        pw = cc % 5 + 5 * dj
        fidx[t] = (ph * 10 + pw) * _MP + (i * _N2 + j)
        wt[t] = (vR[di] / cntR) * (vC[dj] / cntC)
    return fidx.reshape(-1), wt.reshape(-1)


_PATCH_TBL = _build_patch_table().reshape(-1)
_FOLD_FIDX, _FOLD_WT = _build_fold_tables()


def _conv_chain(x_ref, w1_ref, b1_ref, w2_ref, b2_ref, w3_ref, b3_ref,
                mask_ref, out_ref):
    xp = x_ref[0]
    mask = mask_ref[...]

    def conv(inp, w_ref, b_ref, cin):
        wide = jnp.concatenate(
            [jnp.zeros((cin, 128), jnp.float32), inp,
             jnp.zeros((cin, 128), jnp.float32)], axis=1)
        acc = None
        for dy in range(3):
            for dx in range(3):
                s = (dy - 1) * _HP + (dx - 1)
                sl = lax.slice(wide, (0, 128 + s), (cin, 128 + s + _FLATP))
                t = jnp.dot(w_ref[dy * 3 + dx], sl,
                            preferred_element_type=jnp.float32)
                acc = t if acc is None else acc + t
        return acc + b_ref[...]

    y1 = jnp.maximum(conv(xp, w1_ref, b1_ref, 8), 0.0) * mask
    y2 = jnp.maximum(conv(y1, w2_ref, b2_ref, 64), 0.0) * mask
    out_ref[0] = conv(y2, w3_ref, b3_ref, 64)


def _run_convs(x_flat, w1r, b1r, w2r, b2r, w3r, b3r, maskr):
    return pl.pallas_call(
        _conv_chain,
        grid=(2,),
        in_specs=[
            pl.BlockSpec((1, _C, _FLATP), lambda b: (b, 0, 0)),
            pl.BlockSpec((9, 64, 8), lambda b: (0, 0, 0)),
            pl.BlockSpec((64, 1), lambda b: (0, 0)),
            pl.BlockSpec((9, 64, 64), lambda b: (0, 0, 0)),
            pl.BlockSpec((64, 1), lambda b: (0, 0)),
            pl.BlockSpec((9, 8, 64), lambda b: (0, 0, 0)),
            pl.BlockSpec((8, 1), lambda b: (0, 0)),
            pl.BlockSpec((1, _FLATP), lambda b: (0, 0)),
        ],
        out_specs=pl.BlockSpec((1, _C, _FLATP), lambda b: (b, 0, 0)),
        out_shape=jax.ShapeDtypeStruct((2, _C, _FLATP), jnp.float32),
    )(x_flat, w1r, b1r, w2r, b2r, w3r, b3r, maskr)


@functools.lru_cache(maxsize=None)
def _sc_patchify():
    mesh = plsc.VectorSubcoreMesh(core_axis_name="c", subcore_axis_name="s")

    @functools.partial(
        pl.kernel,
        out_type=jax.ShapeDtypeStruct((2, 2, _C, 100 * _MP), jnp.float32),
        mesh=mesh,
        scratch_types=[
            pltpu.VMEM((_FLATP,), jnp.float32),
            pltpu.VMEM((100 * _MP,), jnp.int32),
            pltpu.VMEM((100 * _MP,), jnp.float32),
        ],
        compiler_params=pltpu.CompilerParams(needs_layout_passes=False),
    )
    def _patchify(src_hbm, tbl_hbm, out_hbm, img_v, tbl_v, out_v):
        wid = lax.axis_index("s") * 2 + lax.axis_index("c")
        which = wid // 16
        rem = wid % 16
        b = rem // 8
        c = rem % 8
        pltpu.sync_copy(tbl_hbm, tbl_v)
        pltpu.sync_copy(src_hbm.at[wid], img_v)

        def chunk(t, carry):
            sl = pl.ds(t * 16, 16)
            out_v[sl] = plsc.load_gather(img_v, [tbl_v[sl]])
            return carry

        lax.fori_loop(0, (100 * _MP) // 16, chunk, 0)
        pltpu.sync_copy(out_v, out_hbm.at[which, b, c])

    return _patchify


def _log1p(y):
    u = 1.0 + y
    d = u - 1.0
    return jnp.where(d == 0.0, y, jnp.log(u) * (y / d))


def _expm1(x):
    u = jnp.exp(x)
    d = u - 1.0
    return jnp.where(d == 0.0, x, d * (x / jnp.log(u)))


def _log1mexp(x, guard=1e-7):
    t = x < math.log(0.5)
    x1 = jnp.where(t, x, -1.0)
    x2 = jnp.where(t, -1.0, x)
    y1 = _log1p(-jnp.exp(x1))
    y2 = jnp.log(-_expm1(x2) + guard)
    return jnp.where(t, y1, y2)


def _knn_body(xfT_ref, xefT_ref, lt_ref, z_ref):
    xfT = xfT_ref[0, 0]
    xefT = xefT_ref[0, 0]
    temp = jnp.exp(lt_ref[0, 0])

    xe2 = xefT * xefT
    sq_row = jnp.sum(xe2, axis=0, keepdims=True)
    sq_col = lax.dot_general(xe2, jnp.ones((_F, 1), jnp.float32),
                             (((0,), (0,)), ((), ())),
                             precision=lax.Precision.HIGHEST,
                             preferred_element_type=jnp.float32)
    gram = lax.dot_general(xefT, xefT, (((0,), (0,)), ((), ())),
                           preferred_element_type=jnp.float32)

    dfull = (sq_col + sq_row) - 2.0 * gram
    logits = (-dfull) / temp
    ri = lax.broadcasted_iota(jnp.int32, (_MP, _MP), 0)
    ci = lax.broadcasted_iota(jnp.int32, (_MP, _MP), 1)
    kill = jnp.logical_or(ri == ci, ci >= _M)
    logits = jnp.where(kill, _NEG, logits)

    for k in range(_K):
        mx = jnp.max(logits, axis=1, keepdims=True)
        shifted = logits - mx
        sm = jnp.sum(jnp.exp(shifted), axis=1, keepdims=True)
        w = shifted - jnp.log(sm)
        wk = jnp.exp(w)
        z_ref[0, k] = lax.dot_general(xfT, wk, (((1,), (1,)), ((), ())),
                                      preferred_element_type=jnp.float32)
        if k < _K - 1:
            logits = logits + _log1mexp(w)


def _run_knn(xfT, log_temp2d):
    return pl.pallas_call(
        _knn_body,
        grid=(2,),
        in_specs=[
            pl.BlockSpec((1, 1, _F, _MP), lambda b: (0, b, 0, 0)),
            pl.BlockSpec((1, 1, _F, _MP), lambda b: (1, b, 0, 0)),
            pl.BlockSpec((1, 1), lambda b: (0, 0)),
        ],
        out_specs=pl.BlockSpec((1, _K, _F, _MP), lambda b: (b, 0, 0, 0)),
        out_shape=jax.ShapeDtypeStruct((2, _K, _F, _MP), jnp.float32),
    )(xfT, xfT, log_temp2d)


@functools.lru_cache(maxsize=None)
def _sc_fold():
    mesh = plsc.VectorSubcoreMesh(core_axis_name="c", subcore_axis_name="s")

    @functools.partial(
        pl.kernel,
        out_type=jax.ShapeDtypeStruct((2, _K * _C, _H * _W), jnp.float32),
        mesh=mesh,
        scratch_types=[
            pltpu.VMEM((100 * _MP,), jnp.float32),
            pltpu.VMEM((4 * _H * _W,), jnp.int32),
            pltpu.VMEM((4 * _H * _W,), jnp.float32),
            pltpu.VMEM((_H * _W,), jnp.float32),
        ],
        compiler_params=pltpu.CompilerParams(needs_layout_passes=False),
    )
    def _fold(z_hbm, fidx_hbm, wt_hbm, out_hbm, zblk_v, fidx_v, wt_v, out_v):
        wid = lax.axis_index("s") * 2 + lax.axis_index("c")
        pltpu.sync_copy(fidx_hbm, fidx_v)
        pltpu.sync_copy(wt_hbm, wt_v)
        for u in range(4):
            unit = wid + u * 32
            @pl.when(unit < 2 * _K * _C)
            def _():
                b = unit // (_K * _C)
                rem = unit % (_K * _C)
                k = rem // _C
                c = rem % _C
                pltpu.sync_copy(z_hbm.at[b, k, c], zblk_v)

                def chunk(t, carry):
                    sl = pl.ds(t * 16, 16)
                    acc = jnp.zeros((16,), jnp.float32)
                    for term in range(4):
                        tsl = pl.ds(term * (_H * _W) + t * 16, 16)
                        g = plsc.load_gather(zblk_v, [fidx_v[tsl]])
                        acc = acc + wt_v[tsl] * g
                    out_v[sl] = acc
                    return carry

                lax.fori_loop(0, (_H * _W) // 16, chunk, 0)
                pltpu.sync_copy(out_v, out_hbm.at[b, rem])

    return _fold


def kernel(x, W1, b1, W2, b2, W3, b3, log_temp):
    x = x.astype(jnp.float32)
    xpad = jnp.pad(x, ((0, 0), (0, 0), (1, 1), (1, 1)))
    x_flat = jnp.pad(xpad.reshape(2, _C, _FLAT), ((0, 0), (0, 0), (0, 4)))

    w1r = W1.transpose(2, 3, 0, 1).reshape(9, 64, 8).astype(jnp.float32)
    w2r = W2.transpose(2, 3, 0, 1).reshape(9, 64, 64).astype(jnp.float32)
    w3r = W3.transpose(2, 3, 0, 1).reshape(9, 8, 64).astype(jnp.float32)
    b1r = b1.reshape(64, 1).astype(jnp.float32)
    b2r = b2.reshape(64, 1).astype(jnp.float32)
    b3r = b3.reshape(8, 1).astype(jnp.float32)

    interior = np.zeros((_HP, _HP), np.float32)
    interior[1:-1, 1:-1] = 1.0
    maskr = jnp.asarray(
        np.pad(interior.reshape(-1), (0, 4)).reshape(1, _FLATP))

    xe_flat = _run_convs(x_flat, w1r, b1r, w2r, b2r, w3r, b3r, maskr)

    src = jnp.stack([x_flat, xe_flat]).reshape(32, _FLATP)
    xfT = _sc_patchify()(src, jnp.asarray(_PATCH_TBL))
    xfT = xfT.reshape(2, 2, _F, _MP)

    z = _run_knn(xfT, log_temp.reshape(1, 1).astype(jnp.float32))

    zimg = _sc_fold()(z.reshape(2, _K, _C, 100 * _MP),
                      jnp.asarray(_FOLD_FIDX),
                      jnp.asarray(_FOLD_WT))

    return jnp.concatenate(
        [x, zimg.reshape(2, _K * _C, _H, _W)], axis=1)
